# baseline (speedup 1.0000x reference)
import functools

import jax
import jax.numpy as jnp
from jax import lax
from jax.experimental import pallas as pl
from jax.experimental.pallas import tpu as pltpu
from jax.experimental.pallas import tpu_sc as plsc

N_GENES = 1000000
N_DH = 16
B = 16384

NC = 2
NS = 16
NW = NC * NS
G_PER_TILE = B // NW
GRP = 32


def _height_body(lat_hbm, idx_hbm, tab_hbm, out_hbm, idx_v, win_v, buf_v,
                 lat_v, sem):
    wid = lax.axis_index("s") * NC + lax.axis_index("c")
    base = wid * G_PER_TILE
    pltpu.sync_copy(idx_hbm.at[pl.ds(base, G_PER_TILE)], idx_v)
    pltpu.sync_copy(lat_hbm, lat_v)
    lat = lat_v[...]
    iota = lax.iota(jnp.int32, 16)

    def step(s, carry):
        chunks = [idx_v[pl.ds(s * GRP + h * 16, 16)] for h in range(GRP // 16)]
        copies = []
        for t in range(GRP):
            g = chunks[t // 16][t % 16]
            lane0 = pl.multiple_of((g // 128) * 128, 128)
            copies.append(
                pltpu.async_copy(
                    tab_hbm.at[pl.ds(0, 8), pl.ds(lane0, 128)],
                    win_v.at[t, pl.ds(0, 8)], sem
                )
            )
            copies.append(
                pltpu.async_copy(
                    tab_hbm.at[pl.ds(8, 8), pl.ds(lane0, 128)],
                    win_v.at[t, pl.ds(8, 8)], sem
                )
            )
        for t in range(GRP):
            copies[2 * t].wait()
            copies[2 * t + 1].wait()
            g = chunks[t // 16][t % 16]
            c16 = lax.broadcast(g - (g // 128) * 128, (16,))
            col = plsc.load_gather(win_v.at[t], [iota, c16])
            slot = lax.broadcast(s * GRP + t, (16,))
            plsc.store_scatter(buf_v, [iota, slot], col * lat)
        return carry

    lax.fori_loop(0, G_PER_TILE // GRP, step, 0)
    pltpu.sync_copy(buf_v, out_hbm.at[:, pl.ds(base, G_PER_TILE)])


_height_kernel = pl.kernel(
    _height_body,
    mesh=plsc.VectorSubcoreMesh(core_axis_name="c", subcore_axis_name="s"),
    out_type=jax.ShapeDtypeStruct((N_DH, B), jnp.float32),
    scratch_types=[
        pltpu.VMEM((G_PER_TILE,), jnp.int32),
        pltpu.VMEM((GRP, N_DH, 128), jnp.float32),
        pltpu.VMEM((N_DH, G_PER_TILE), jnp.float32),
        pltpu.VMEM((N_DH,), jnp.float32),
        pltpu.SemaphoreType.DMA,
    ],
    compiler_params=pltpu.CompilerParams(
        use_tc_tiling_on_sc=True, needs_layout_passes=False
    ),
)


OVERALL_BLK = 131072


def _overall_body(lat_ref, w_ref, out_ref):
    out_ref[...] = lat_ref[...] * w_ref[...]


def _overall(latm, w_flat):
    grid = (N_GENES + OVERALL_BLK - 1) // OVERALL_BLK
    return pl.pallas_call(
        _overall_body,
        grid=(grid,),
        in_specs=[
            pl.BlockSpec((N_DH, 1), lambda i: (0, 0)),
            pl.BlockSpec((1, OVERALL_BLK), lambda i: (0, i)),
        ],
        out_specs=pl.BlockSpec((N_DH, OVERALL_BLK), lambda i: (0, i)),
        out_shape=jax.ShapeDtypeStruct((N_DH, N_GENES), jnp.float32),
    )(latm, w_flat)


def kernel(latent, genes_oi, W_height, W_overall):
    height_t = _height_kernel(latent, genes_oi, W_height.T)
    overall_t = _overall(latent.reshape(N_DH, 1), W_overall.T)
    return (height_t.T, overall_t.T)


# speedup vs baseline: 1.0149x; 1.0149x over previous
import functools

import jax
import jax.numpy as jnp
from jax import lax
from jax.experimental import pallas as pl
from jax.experimental.pallas import tpu as pltpu
from jax.experimental.pallas import tpu_sc as plsc

N_GENES = 1000000
N_DH = 16
B = 16384

NC = 2
NS = 16
NW = NC * NS
G_PER_TILE = B // NW
GRP = 32


def _height_body(lat_hbm, idx_hbm, tab_hbm, out_hbm, idx_v, win_v, buf_v,
                 lat_v, sem):
    wid = lax.axis_index("s") * NC + lax.axis_index("c")
    base = wid * G_PER_TILE
    pltpu.sync_copy(idx_hbm.at[pl.ds(base, G_PER_TILE)], idx_v)
    pltpu.sync_copy(lat_hbm, lat_v)
    lat = lat_v[...]
    iota = lax.iota(jnp.int32, 16)

    def step(s, carry):
        chunks = [idx_v[pl.ds(s * GRP + h * 16, 16)] for h in range(GRP // 16)]
        copies = []
        for t in range(GRP):
            g = chunks[t // 16][t % 16]
            lane0 = pl.multiple_of((g // 128) * 128, 128)
            copies.append(
                pltpu.async_copy(
                    tab_hbm.at[:, pl.ds(lane0, 128)], win_v.at[t], sem
                )
            )
        for t in range(GRP):
            copies[t].wait()
            g = chunks[t // 16][t % 16]
            c16 = lax.broadcast(g - (g // 128) * 128, (16,))
            col = plsc.load_gather(win_v.at[t], [iota, c16])
            slot = lax.broadcast(s * GRP + t, (16,))
            plsc.store_scatter(buf_v, [iota, slot], col * lat)
        return carry

    lax.fori_loop(0, G_PER_TILE // GRP, step, 0)
    pltpu.sync_copy(buf_v, out_hbm.at[:, pl.ds(base, G_PER_TILE)])


_height_kernel = pl.kernel(
    _height_body,
    mesh=plsc.VectorSubcoreMesh(core_axis_name="c", subcore_axis_name="s"),
    out_type=jax.ShapeDtypeStruct((N_DH, B), jnp.float32),
    scratch_types=[
        pltpu.VMEM((G_PER_TILE,), jnp.int32),
        pltpu.VMEM((GRP, N_DH, 128), jnp.float32),
        pltpu.VMEM((N_DH, G_PER_TILE), jnp.float32),
        pltpu.VMEM((N_DH,), jnp.float32),
        pltpu.SemaphoreType.DMA,
    ],
    compiler_params=pltpu.CompilerParams(
        use_tc_tiling_on_sc=True, needs_layout_passes=False
    ),
)


OVERALL_BLK = 131072


def _overall_body(lat_ref, w_ref, out_ref):
    out_ref[...] = lat_ref[...] * w_ref[...]


def _overall(latm, w_flat):
    grid = (N_GENES + OVERALL_BLK - 1) // OVERALL_BLK
    return pl.pallas_call(
        _overall_body,
        grid=(grid,),
        in_specs=[
            pl.BlockSpec((N_DH, 1), lambda i: (0, 0)),
            pl.BlockSpec((1, OVERALL_BLK), lambda i: (0, i)),
        ],
        out_specs=pl.BlockSpec((N_DH, OVERALL_BLK), lambda i: (0, i)),
        out_shape=jax.ShapeDtypeStruct((N_DH, N_GENES), jnp.float32),
    )(latm, w_flat)


def kernel(latent, genes_oi, W_height, W_overall):
    height_t = _height_kernel(latent, genes_oi, W_height.T)
    overall_t = _overall(latent.reshape(N_DH, 1), W_overall.T)
    return (height_t.T, overall_t.T)
